# Initial kernel scaffold; baseline (speedup 1.0000x reference)
import jax
import jax.numpy as jnp
from jax.experimental import pallas as pl


def kernel(feats, W1, gamma1, beta1, W2, gamma2, beta2, in_idx, out_idx, valid):
    raise NotImplementedError("write your pallas kernel here")


# speedup vs baseline: 9.9641x; 9.9641x over previous
import functools

import jax
import jax.numpy as jnp
from jax import lax
from jax.experimental import pallas as pl
from jax.experimental.pallas import tpu as pltpu
from jax.experimental.pallas import tpu_sc as plsc

_LANES = 16
_NW = 32


def _wid():
    return lax.axis_index("s") * 2 + lax.axis_index("c")


@functools.lru_cache(maxsize=None)
def _build_invert(K, N):
    KN = K * N
    C = 2000
    assert N % C == 0 and C % _LANES == 0 and N % _LANES == 0
    mesh = plsc.VectorSubcoreMesh(core_axis_name="c", subcore_axis_name="s")

    def body(outf, inf, valf, g, buf, oc, ic, vc):
        w = _wid()

        @pl.when(w < K)
        def _():
            lanes = jnp.arange(_LANES, dtype=jnp.int32)

            def init(i, carry):
                buf[pl.ds(i * _LANES, _LANES)] = lanes + (KN + i * _LANES)
                return carry

            lax.fori_loop(0, N // _LANES, init, 0)
            base = w * N

            def chunk(c, carry):
                off = base + c * C
                pltpu.sync_copy(outf.at[pl.ds(off, C)], oc)
                pltpu.sync_copy(inf.at[pl.ds(off, C)], ic)
                pltpu.sync_copy(valf.at[pl.ds(off, C)], vc)

                def scat(j, carry2):
                    s = pl.ds(j * _LANES, _LANES)
                    plsc.store_scatter(buf, [oc[s]], ic[s] + base,
                                       mask=vc[s] > 0.0)
                    return carry2

                lax.fori_loop(0, C // _LANES, scat, 0)
                return carry

            lax.fori_loop(0, N // C, chunk, 0)
            pltpu.sync_copy(buf, g.at[pl.ds(base, N)])

    return pl.kernel(
        body,
        out_type=jax.ShapeDtypeStruct((KN,), jnp.int32),
        mesh=mesh,
        compiler_params=pltpu.CompilerParams(needs_layout_passes=False),
        scratch_types=[
            pltpu.VMEM((N,), jnp.int32),
            pltpu.VMEM((C,), jnp.int32),
            pltpu.VMEM((C,), jnp.int32),
            pltpu.VMEM((C,), jnp.float32),
        ],
    )


@functools.lru_cache(maxsize=None)
def _build_gather_accum(K, N, CH):
    R = 512
    NCH = -(-N // R)
    MAXIT = -(-NCH // _NW)
    assert N >= R and N % 8 == 0 and R % 8 == 0
    mesh = plsc.VectorSubcoreMesh(core_axis_name="c", subcore_axis_name="s")

    def body(z, g, h, idxv, acc, semi, semg):
        w = _wid()

        def chunk(it, carry):
            cid = w + it * _NW

            @pl.when(cid < NCH)
            def _():
                start = jnp.minimum(cid * R, N - R)

                def fire_i(kk, c2):
                    pltpu.make_async_copy(
                        g.at[pl.ds(kk * N + start, R)], idxv.at[pl.ds(kk * R, R)], semi
                    ).start()
                    return c2

                lax.fori_loop(0, K, fire_i, 0)

                def drain_i(kk, c2):
                    pltpu.make_async_copy(
                        g.at[pl.ds(kk * N + start, R)], idxv.at[pl.ds(kk * R, R)], semi
                    ).wait()
                    return c2

                lax.fori_loop(0, K, drain_i, 0)

                cp0 = pltpu.make_async_copy(z.at[idxv.at[pl.ds(0, R)]], acc, semg)
                cp0.start()
                cp0.wait()

                def fire_k(kk, c2):
                    pltpu.async_copy(z.at[idxv.at[pl.ds(kk * R, R)]], acc, semg,
                                     add=True)
                    return c2

                lax.fori_loop(1, K, fire_k, 0)

                def drain_k(kk, c2):
                    pltpu.make_async_copy(z.at[idxv.at[pl.ds(kk * R, R)]], acc,
                                          semg).wait()
                    return c2

                lax.fori_loop(1, K, drain_k, 0)
                pltpu.sync_copy(acc, h.at[pl.ds(start, R)])

            return carry

        lax.fori_loop(0, MAXIT, chunk, 0)

    return pl.kernel(
        body,
        out_type=jax.ShapeDtypeStruct((N, CH), jnp.float32),
        mesh=mesh,
        scratch_types=[
            pltpu.VMEM((K * R,), jnp.int32),
            pltpu.VMEM((R, CH), jnp.float32),
            pltpu.SemaphoreType.DMA,
            pltpu.SemaphoreType.DMA,
        ],
    )


def _tap_matmul(x, W, a=None, b=None):
    N, CIN = x.shape
    K, _, COUT = W.shape
    TILE = 2000
    assert N % TILE == 0
    T = N // TILE
    affine = a is not None

    def body(*refs):
        if affine:
            x_ref, w_ref, a_ref, b_ref, o_ref = refs
        else:
            x_ref, w_ref, o_ref = refs
        k = pl.program_id(1)
        xv = x_ref[...]
        if affine:
            xv = jnp.maximum(xv * a_ref[...] + b_ref[...], 0.0)
        r = jnp.dot(xv, w_ref[0], preferred_element_type=jnp.float32)
        o_ref[...] = jnp.where(k < K, r, 0.0)

    in_specs = [
        pl.BlockSpec((TILE, CIN), lambda t, k: (t, 0)),
        pl.BlockSpec((1, CIN, COUT), lambda t, k: (jnp.minimum(k, K - 1), 0, 0)),
    ]
    args = [x, W]
    if affine:
        in_specs += [
            pl.BlockSpec((1, CIN), lambda t, k: (0, 0)),
            pl.BlockSpec((1, CIN), lambda t, k: (0, 0)),
        ]
        args += [a, b]
    return pl.pallas_call(
        body,
        grid=(T, K + 1),
        in_specs=in_specs,
        out_specs=pl.BlockSpec((TILE, COUT), lambda t, k: (k * T + t, 0)),
        out_shape=jax.ShapeDtypeStruct(((K + 1) * N, COUT), jnp.float32),
    )(*args)


def _col_stats(x):
    N, CH = x.shape
    TILE = 2000
    T = N // TILE

    def body(x_ref, o_ref):
        @pl.when(pl.program_id(0) == 0)
        def _():
            o_ref[...] = jnp.zeros_like(o_ref)

        xv = x_ref[...]
        o_ref[0:1, :] += jnp.sum(xv, axis=0, keepdims=True)
        o_ref[1:2, :] += jnp.sum(xv * xv, axis=0, keepdims=True)

    return pl.pallas_call(
        body,
        grid=(T,),
        in_specs=[pl.BlockSpec((TILE, CH), lambda t: (t, 0))],
        out_specs=pl.BlockSpec((8, CH), lambda t: (0, 0)),
        out_shape=jax.ShapeDtypeStruct((8, CH), jnp.float32),
    )(x)


def _residual_epilogue(h, a, b, res):
    N, CH = h.shape
    TILE = 2000
    T = N // TILE

    def body(h_ref, a_ref, b_ref, r_ref, o_ref):
        o_ref[...] = jnp.maximum(
            h_ref[...] * a_ref[...] + b_ref[...] + r_ref[...], 0.0)

    return pl.pallas_call(
        body,
        grid=(T,),
        in_specs=[
            pl.BlockSpec((TILE, CH), lambda t: (t, 0)),
            pl.BlockSpec((1, CH), lambda t: (0, 0)),
            pl.BlockSpec((1, CH), lambda t: (0, 0)),
            pl.BlockSpec((TILE, CH), lambda t: (t, 0)),
        ],
        out_specs=pl.BlockSpec((TILE, CH), lambda t: (t, 0)),
        out_shape=jax.ShapeDtypeStruct((N, CH), jnp.float32),
    )(h, a, b, res)


def _bn_affine(stats, gamma, beta, n, eps=1e-5):
    mu = stats[0] / n
    var = stats[1] / n - mu * mu
    a = gamma * lax.rsqrt(var + eps)
    b = beta - mu * a
    return a.reshape(1, -1), b.reshape(1, -1)


def kernel(feats, W1, gamma1, beta1, W2, gamma2, beta2, in_idx, out_idx, valid):
    N, _ = feats.shape
    K = W1.shape[0]

    in_f = in_idx.reshape(-1).astype(jnp.int32)
    out_f = out_idx.reshape(-1).astype(jnp.int32)
    val_f = valid.reshape(-1).astype(jnp.float32)

    g = _build_invert(K, N)(out_f, in_f, val_f)
    gacc = _build_gather_accum(K, N, feats.shape[1])

    z1 = _tap_matmul(feats, W1)
    h1 = gacc(z1, g)
    a1, b1 = _bn_affine(_col_stats(h1), gamma1, beta1, N)

    z2 = _tap_matmul(h1, W2, a1, b1)
    h2 = gacc(z2, g)
    a2, b2 = _bn_affine(_col_stats(h2), gamma2, beta2, N)

    return _residual_epilogue(h2, a2, b2, feats)
